# Initial kernel scaffold; baseline (speedup 1.0000x reference)
import functools

import jax
import jax.numpy as jnp
from jax import lax
from jax.experimental import pallas as pl
from jax.experimental.pallas import tpu as pltpu
from jax.experimental.pallas import tpu_sc as plsc

N_NODES = 100000
N_EDGES = 1600000
NWORK = 32
EPW = N_EDGES // NWORK
RPB = 1024
NB = 98
NPAD = NB * RPB
CAP = 768
SEG = NB * CAP
WIN = 2000
BG = 128

_mesh = plsc.VectorSubcoreMesh(core_axis_name="c", subcore_axis_name="s")


@functools.partial(
    pl.kernel,
    mesh=_mesh,
    out_type=[
        jax.ShapeDtypeStruct((NWORK, SEG), jnp.int32),
        jax.ShapeDtypeStruct((NWORK, 128), jnp.int32),
    ],
    scratch_types=[
        pltpu.VMEM((SEG,), jnp.int32),
        pltpu.VMEM((WIN,), jnp.int32),
        pltpu.VMEM((WIN,), jnp.int32),
        pltpu.VMEM((128,), jnp.int32),
    ],
)
def _bin_edges(ei_hbm, binned_hbm, counts_hbm, stage_v, src_v, dst_v, cnt_v):
    w = lax.axis_index("s") * 2 + lax.axis_index("c")
    base_e = w * EPW

    def zero_body(i, carry):
        cnt_v[pl.ds(i * 16, 16)] = jnp.zeros((16,), jnp.int32)
        return carry

    lax.fori_loop(0, 8, zero_body, 0)

    def win_body(iw, carry):
        off = base_e + iw * WIN
        pltpu.sync_copy(ei_hbm.at[pl.ds(off, WIN)], src_v)
        pltpu.sync_copy(ei_hbm.at[pl.ds(N_EDGES + off, WIN)], dst_v)

        def e_body(e, c2):
            d = dst_v[e]
            s = src_v[e]
            b = lax.shift_right_logical(d, 10)
            c = cnt_v[b]
            safe = jnp.minimum(c, CAP - 1)
            stage_v[b * CAP + safe] = s * 1024 + jnp.bitwise_and(d, 1023)
            cnt_v[b] = jnp.minimum(c + 1, CAP)
            return c2

        lax.fori_loop(0, WIN, e_body, 0)
        return carry

    lax.fori_loop(0, EPW // WIN, win_body, 0)
    pltpu.sync_copy(stage_v, binned_hbm.at[w])
    pltpu.sync_copy(cnt_v, counts_hbm.at[w])


def _make_segmax(C):
    @functools.partial(
        pl.kernel,
        mesh=_mesh,
        out_type=jax.ShapeDtypeStruct((NPAD * C,), jnp.float32),
        scratch_types=[
            pltpu.VMEM((RPB * C,), jnp.float32),
            pltpu.VMEM((BG,), jnp.int32),
            pltpu.VMEM((BG,), jnp.int32),
            pltpu.VMEM((BG, C), jnp.float32),
            pltpu.VMEM((NWORK, 128), jnp.int32),
            pltpu.SemaphoreType.DMA,
        ],
    )
    def segmax(h_hbm, binned_hbm, counts_hbm, m_hbm,
               acc_v, pk_v, idx_v, rows_v, cnt_v, sem):
        w = lax.axis_index("s") * 2 + lax.axis_index("c")
        pltpu.sync_copy(counts_hbm, cnt_v)
        ninf = jnp.full((16,), -jnp.inf, dtype=jnp.float32)

        for r in range(4):
            b = r * NWORK + w

            @pl.when(b < NB)
            def _():
                def init_body(i, carry):
                    acc_v[pl.ds(i * 16, 16)] = ninf
                    return carry

                lax.fori_loop(0, RPB * C // 16, init_body, 0)

                def w2_body(w2, carry):
                    nw = cnt_v[w2, b]
                    nbatch = lax.div(nw + BG - 1, BG)

                    def s_body(s, c2):
                        pltpu.sync_copy(
                            binned_hbm.at[w2, pl.ds(b * CAP + s * BG, BG)],
                            pk_v)

                        def up_body(i, c3):
                            v = pk_v[pl.ds(i * 16, 16)]
                            idx_v[pl.ds(i * 16, 16)] = jnp.minimum(
                                lax.shift_right_logical(v, 10), N_NODES - 1)
                            return c3

                        lax.fori_loop(0, BG // 16, up_body, 0)
                        pltpu.async_copy(h_hbm.at[idx_v], rows_v, sem).wait()
                        lim = jnp.minimum(BG, nw - s * BG)

                        def e_body(e, c4):
                            pk = pk_v[e]
                            base = jnp.bitwise_and(pk, 1023) * C
                            for cc in range(C // 16):
                                o = base + cc * 16
                                acc_v[pl.ds(o, 16)] = jnp.maximum(
                                    acc_v[pl.ds(o, 16)],
                                    rows_v[e, pl.ds(cc * 16, 16)])
                            return c4

                        lax.fori_loop(0, lim, e_body, 0)
                        return c2

                    lax.fori_loop(0, nbatch, s_body, 0)
                    return carry

                lax.fori_loop(0, NWORK, w2_body, 0)
                pltpu.sync_copy(acc_v, m_hbm.at[pl.ds(b * RPB * C, RPB * C)])

    return segmax


_segmax16 = _make_segmax(16)
_segmax64 = _make_segmax(64)

_BLK = 1000


def _tc_layer_body(m_ref, h_ref, w_ref, b_ref, o_ref):
    m = m_ref[...]
    agg = jnp.where(m > -jnp.inf, m - h_ref[...], 0.0)
    y = jnp.dot(agg, w_ref[...], preferred_element_type=jnp.float32)
    o_ref[...] = jnp.maximum(y + b_ref[...], 0.0)


def _tc_layer(m, h, W, b):
    cin, cout = W.shape
    return pl.pallas_call(
        _tc_layer_body,
        grid=(N_NODES // _BLK,),
        in_specs=[
            pl.BlockSpec((_BLK, cin), lambda i: (i, 0)),
            pl.BlockSpec((_BLK, cin), lambda i: (i, 0)),
            pl.BlockSpec((cin, cout), lambda i: (0, 0)),
            pl.BlockSpec((1, cout), lambda i: (0, 0)),
        ],
        out_specs=pl.BlockSpec((_BLK, cout), lambda i: (i, 0)),
        out_shape=jax.ShapeDtypeStruct((N_NODES, cout), jnp.float32),
    )(m, h, W, b.reshape(1, cout))


def _tc_final_body(m_ref, h_ref, w_ref, b_ref, wo_ref, bo_ref, o_ref):
    m = m_ref[...]
    agg = jnp.where(m > -jnp.inf, m - h_ref[...], 0.0)
    y = jnp.dot(agg, w_ref[...], preferred_element_type=jnp.float32)
    h3 = jnp.maximum(y + b_ref[...], 0.0)
    s = jnp.dot(h3, wo_ref[...], preferred_element_type=jnp.float32)
    o_ref[...] = jax.nn.sigmoid(s + bo_ref[...])


def _tc_final(m, h, W, b, Wout, bout):
    return pl.pallas_call(
        _tc_final_body,
        grid=(N_NODES // _BLK,),
        in_specs=[
            pl.BlockSpec((_BLK, 64), lambda i: (i, 0)),
            pl.BlockSpec((_BLK, 64), lambda i: (i, 0)),
            pl.BlockSpec((64, 64), lambda i: (0, 0)),
            pl.BlockSpec((1, 64), lambda i: (0, 0)),
            pl.BlockSpec((64, 1), lambda i: (0, 0)),
            pl.BlockSpec((1, 1), lambda i: (0, 0)),
        ],
        out_specs=pl.BlockSpec((_BLK, 1), lambda i: (i, 0)),
        out_shape=jax.ShapeDtypeStruct((N_NODES, 1), jnp.float32),
    )(m, h, W, b.reshape(1, 64), Wout, bout.reshape(1, 1))


def kernel(x, edge_index, W1, b1, W2, b2, W3, b3, Wout, bout):
    ei = edge_index.reshape(-1)
    x16 = jnp.pad(x, ((0, 0), (0, 13)))
    W1p = jnp.pad(W1, ((0, 13), (0, 0)))

    binned, counts = _bin_edges(ei)

    m1 = _segmax16(x16, binned, counts).reshape(NPAD, 16)[:N_NODES]
    h1 = _tc_layer(m1, x16, W1p, b1)

    m2 = _segmax64(h1, binned, counts).reshape(NPAD, 64)[:N_NODES]
    h2 = _tc_layer(m2, h1, W2, b2)

    m3 = _segmax64(h2, binned, counts).reshape(NPAD, 64)[:N_NODES]
    out = _tc_final(m3, h2, W3, b3, Wout, bout)
    return out[:, 0]


# speedup vs baseline: 3.5996x; 3.5996x over previous
import functools

import jax
import jax.numpy as jnp
from jax import lax
from jax.experimental import pallas as pl
from jax.experimental.pallas import tpu as pltpu
from jax.experimental.pallas import tpu_sc as plsc

N_NODES = 100000
N_EDGES = 1600000
NWORK = 32
EPW = N_EDGES // NWORK
RPB = 1024
NB = 98
NPAD = NB * RPB
CAP = 768
SEG = NB * CAP
WIN = 2000
BG = 128

_mesh = plsc.VectorSubcoreMesh(core_axis_name="c", subcore_axis_name="s")


@functools.partial(
    pl.kernel,
    mesh=_mesh,
    out_type=[
        jax.ShapeDtypeStruct((NWORK, SEG), jnp.int32),
        jax.ShapeDtypeStruct((NWORK, 128), jnp.int32),
    ],
    scratch_types=[
        pltpu.VMEM((SEG,), jnp.int32),
        pltpu.VMEM((WIN,), jnp.int32),
        pltpu.VMEM((WIN,), jnp.int32),
        pltpu.VMEM((128,), jnp.int32),
    ],
    compiler_params=pltpu.CompilerParams(needs_layout_passes=False, use_tc_tiling_on_sc=False),
)
def _bin_edges(ei_hbm, binned_hbm, counts_hbm, stage_v, src_v, dst_v, cnt_v):
    w = lax.axis_index("s") * 2 + lax.axis_index("c")
    base_e = w * EPW
    iota = lax.iota(jnp.int32, 16)

    def zero_body(i, carry):
        cnt_v[pl.ds(i * 16, 16)] = jnp.zeros((16,), jnp.int32)
        return carry

    lax.fori_loop(0, 8, zero_body, 0)

    def win_body(iw, carry):
        off = base_e + iw * WIN
        pltpu.sync_copy(ei_hbm.at[pl.ds(off, WIN)], src_v)
        pltpu.sync_copy(ei_hbm.at[pl.ds(N_EDGES + off, WIN)], dst_v)

        def g_body(g, c2):
            d = dst_v[pl.ds(g * 16, 16)]
            s = src_v[pl.ds(g * 16, 16)]
            b16 = lax.shift_right_logical(d, 10)
            pk16 = s * 1024 + jnp.bitwise_and(d, 1023)
            bs, perm = plsc.sort_key_val(b16, iota)
            pks = jnp.take_along_axis(pk16, perm, axis=0)
            prev = jnp.take_along_axis(bs, jnp.maximum(iota - 1, 0), axis=0)
            is_start = (bs != prev) | (iota == 0)
            start_pos = plsc.cummax(jnp.where(is_start, iota, 0))
            rank = iota - start_pos
            cur = plsc.load_gather(cnt_v, [bs])
            pos = jnp.minimum(cur + rank, CAP - 1)
            plsc.store_scatter(stage_v, [bs * CAP + pos], pks)
            nxt = jnp.take_along_axis(bs, jnp.minimum(iota + 1, 15), axis=0)
            is_end = (bs != nxt) | (iota == 15)
            plsc.store_scatter(cnt_v, [bs],
                               jnp.minimum(cur + rank + 1, CAP),
                               mask=is_end)
            return c2

        lax.fori_loop(0, WIN // 16, g_body, 0)
        return carry

    lax.fori_loop(0, EPW // WIN, win_body, 0)
    pltpu.sync_copy(stage_v, binned_hbm.at[w])
    pltpu.sync_copy(cnt_v, counts_hbm.at[w])


def _make_segmax(C):
    @functools.partial(
        pl.kernel,
        mesh=_mesh,
        out_type=jax.ShapeDtypeStruct((NPAD * C,), jnp.float32),
        scratch_types=[
            pltpu.VMEM((RPB * C + C,), jnp.float32),
            pltpu.VMEM((BG,), jnp.int32),
            pltpu.VMEM((BG,), jnp.int32),
            pltpu.VMEM((BG, C), jnp.float32),
            pltpu.VMEM((NWORK, 128), jnp.int32),
            pltpu.SMEM((4 * NWORK,), jnp.int32),
            pltpu.SemaphoreType.DMA,
        ],
        compiler_params=pltpu.CompilerParams(needs_layout_passes=False, use_tc_tiling_on_sc=False),
    )
    def segmax(h_hbm, binned_hbm, counts_hbm, m_hbm,
               acc_v, pk_v, idx_v, rows_v, cnt_v, cnt_s, sem):
        w = lax.axis_index("s") * 2 + lax.axis_index("c")
        pltpu.sync_copy(counts_hbm, cnt_v)
        ninf = jnp.full((16,), -jnp.inf, dtype=jnp.float32)
        iota = lax.iota(jnp.int32, 16)

        for r in range(4):
            b = r * NWORK + w

            @pl.when(b < NB)
            def _():
                bvec = jnp.full((16,), b, dtype=jnp.int32)
                ca = plsc.load_gather(cnt_v, [iota, bvec])
                cb = plsc.load_gather(cnt_v, [iota + 16, bvec])
                for l in range(16):
                    cnt_s[r * NWORK + l] = ca[l]
                    cnt_s[r * NWORK + 16 + l] = cb[l]

                def init_body(i, carry):
                    acc_v[pl.ds(i * 16, 16)] = ninf
                    return carry

                lax.fori_loop(0, RPB * C // 16, init_body, 0)

                def w2_body(w2, carry):
                    nw = cnt_s[r * NWORK + w2]
                    nbatch = lax.shift_right_logical(nw + BG - 1, 7)

                    def s_body(s, c2):
                        pltpu.sync_copy(
                            binned_hbm.at[w2, pl.ds(b * CAP + s * BG, BG)],
                            pk_v)

                        def up_body(i, c3):
                            v = pk_v[pl.ds(i * 16, 16)]
                            idx_v[pl.ds(i * 16, 16)] = jnp.minimum(
                                lax.shift_right_logical(v, 10), N_NODES - 1)
                            return c3

                        lax.fori_loop(0, BG // 16, up_body, 0)
                        pltpu.async_copy(h_hbm.at[idx_v], rows_v, sem).wait()

                        def g_body(g, c4):
                            pk = pk_v[pl.ds(g * 16, 16)]
                            evec = s * BG + g * 16 + iota
                            valid = evec < nw
                            base16 = jnp.where(
                                valid, jnp.bitwise_and(pk, 1023) * C, RPB * C)
                            for l in range(16):
                                base = base16[l]
                                for cc in range(C // 16):
                                    o = base + cc * 16
                                    acc_v[pl.ds(o, 16)] = jnp.maximum(
                                        acc_v[pl.ds(o, 16)],
                                        rows_v[g * 16 + l, pl.ds(cc * 16, 16)])
                            return c4

                        lax.fori_loop(0, BG // 16, g_body, 0)
                        return c2

                    lax.fori_loop(0, nbatch, s_body, 0)
                    return carry

                lax.fori_loop(0, NWORK, w2_body, 0)
                pltpu.sync_copy(acc_v.at[pl.ds(0, RPB * C)],
                                m_hbm.at[pl.ds(b * RPB * C, RPB * C)])

    return segmax


_segmax16 = _make_segmax(16)
_segmax64 = _make_segmax(64)

_BLK = 1000


def _tc_layer_body(m_ref, h_ref, w_ref, b_ref, o_ref):
    m = m_ref[...]
    agg = jnp.where(m > -jnp.inf, m - h_ref[...], 0.0)
    y = jnp.dot(agg, w_ref[...], preferred_element_type=jnp.float32)
    o_ref[...] = jnp.maximum(y + b_ref[...], 0.0)


def _tc_layer(m, h, W, b):
    cin, cout = W.shape
    return pl.pallas_call(
        _tc_layer_body,
        grid=(N_NODES // _BLK,),
        in_specs=[
            pl.BlockSpec((_BLK, cin), lambda i: (i, 0)),
            pl.BlockSpec((_BLK, cin), lambda i: (i, 0)),
            pl.BlockSpec((cin, cout), lambda i: (0, 0)),
            pl.BlockSpec((1, cout), lambda i: (0, 0)),
        ],
        out_specs=pl.BlockSpec((_BLK, cout), lambda i: (i, 0)),
        out_shape=jax.ShapeDtypeStruct((N_NODES, cout), jnp.float32),
    )(m, h, W, b.reshape(1, cout))


def _tc_final_body(m_ref, h_ref, w_ref, b_ref, wo_ref, bo_ref, o_ref):
    m = m_ref[...]
    agg = jnp.where(m > -jnp.inf, m - h_ref[...], 0.0)
    y = jnp.dot(agg, w_ref[...], preferred_element_type=jnp.float32)
    h3 = jnp.maximum(y + b_ref[...], 0.0)
    s = jnp.dot(h3, wo_ref[...], preferred_element_type=jnp.float32)
    o_ref[...] = jax.nn.sigmoid(s + bo_ref[...])


def _tc_final(m, h, W, b, Wout, bout):
    return pl.pallas_call(
        _tc_final_body,
        grid=(N_NODES // _BLK,),
        in_specs=[
            pl.BlockSpec((_BLK, 64), lambda i: (i, 0)),
            pl.BlockSpec((_BLK, 64), lambda i: (i, 0)),
            pl.BlockSpec((64, 64), lambda i: (0, 0)),
            pl.BlockSpec((1, 64), lambda i: (0, 0)),
            pl.BlockSpec((64, 1), lambda i: (0, 0)),
            pl.BlockSpec((1, 1), lambda i: (0, 0)),
        ],
        out_specs=pl.BlockSpec((_BLK, 1), lambda i: (i, 0)),
        out_shape=jax.ShapeDtypeStruct((N_NODES, 1), jnp.float32),
    )(m, h, W, b.reshape(1, 64), Wout, bout.reshape(1, 1))


def kernel(x, edge_index, W1, b1, W2, b2, W3, b3, Wout, bout):
    ei = edge_index.reshape(-1)
    x16 = jnp.pad(x, ((0, 0), (0, 13)))
    W1p = jnp.pad(W1, ((0, 13), (0, 0)))

    binned, counts = _bin_edges(ei)

    m1 = _segmax16(x16, binned, counts).reshape(NPAD, 16)[:N_NODES]
    h1 = _tc_layer(m1, x16, W1p, b1)

    m2 = _segmax64(h1, binned, counts).reshape(NPAD, 64)[:N_NODES]
    h2 = _tc_layer(m2, h1, W2, b2)

    m3 = _segmax64(h2, binned, counts).reshape(NPAD, 64)[:N_NODES]
    out = _tc_final(m3, h2, W3, b3, Wout, bout)
    return out[:, 0]


# speedup vs baseline: 3.9639x; 1.1012x over previous
import functools

import jax
import jax.numpy as jnp
from jax import lax
from jax.experimental import pallas as pl
from jax.experimental.pallas import tpu as pltpu
from jax.experimental.pallas import tpu_sc as plsc

N_NODES = 100000
N_EDGES = 1600000
NWORK = 32
EPW = N_EDGES // NWORK
RPB = 1024
NB = 98
NPAD = NB * RPB
CAP = 768
SEG = NB * CAP
WIN = 2000
BG = 128

_mesh = plsc.VectorSubcoreMesh(core_axis_name="c", subcore_axis_name="s")


@functools.partial(
    pl.kernel,
    mesh=_mesh,
    out_type=[
        jax.ShapeDtypeStruct((NWORK, SEG), jnp.int32),
        jax.ShapeDtypeStruct((NWORK, 128), jnp.int32),
    ],
    scratch_types=[
        pltpu.VMEM((SEG,), jnp.int32),
        pltpu.VMEM((WIN,), jnp.int32),
        pltpu.VMEM((WIN,), jnp.int32),
        pltpu.VMEM((128,), jnp.int32),
    ],
    compiler_params=pltpu.CompilerParams(needs_layout_passes=False, use_tc_tiling_on_sc=False),
)
def _bin_edges(ei_hbm, binned_hbm, counts_hbm, stage_v, src_v, dst_v, cnt_v):
    w = lax.axis_index("s") * 2 + lax.axis_index("c")
    base_e = w * EPW
    iota = lax.iota(jnp.int32, 16)

    def zero_body(i, carry):
        cnt_v[pl.ds(i * 16, 16)] = jnp.zeros((16,), jnp.int32)
        return carry

    lax.fori_loop(0, 8, zero_body, 0)

    def win_body(iw, carry):
        off = base_e + iw * WIN
        pltpu.sync_copy(ei_hbm.at[pl.ds(off, WIN)], src_v)
        pltpu.sync_copy(ei_hbm.at[pl.ds(N_EDGES + off, WIN)], dst_v)

        def g_body(g, c2):
            d = dst_v[pl.ds(g * 16, 16)]
            s = src_v[pl.ds(g * 16, 16)]
            b16 = lax.shift_right_logical(d, 10)
            pk16 = s * 1024 + jnp.bitwise_and(d, 1023)
            bs, perm = plsc.sort_key_val(b16, iota)
            pks = jnp.take_along_axis(pk16, perm, axis=0)
            prev = jnp.take_along_axis(bs, jnp.maximum(iota - 1, 0), axis=0)
            is_start = (bs != prev) | (iota == 0)
            start_pos = plsc.cummax(jnp.where(is_start, iota, 0))
            rank = iota - start_pos
            cur = plsc.load_gather(cnt_v, [bs])
            pos = jnp.minimum(cur + rank, CAP - 1)
            plsc.store_scatter(stage_v, [bs * CAP + pos], pks)
            nxt = jnp.take_along_axis(bs, jnp.minimum(iota + 1, 15), axis=0)
            is_end = (bs != nxt) | (iota == 15)
            plsc.store_scatter(cnt_v, [bs],
                               jnp.minimum(cur + rank + 1, CAP),
                               mask=is_end)
            return c2

        lax.fori_loop(0, WIN // 16, g_body, 0)
        return carry

    lax.fori_loop(0, EPW // WIN, win_body, 0)
    pltpu.sync_copy(stage_v, binned_hbm.at[w])
    pltpu.sync_copy(cnt_v, counts_hbm.at[w])


_MAXB = 200


def _make_segmax(C):
    @functools.partial(
        pl.kernel,
        mesh=_mesh,
        out_type=jax.ShapeDtypeStruct((NPAD * C,), jnp.float32),
        scratch_types=[
            pltpu.VMEM((RPB * C + C,), jnp.float32),
            pltpu.VMEM((3, BG), jnp.int32),
            pltpu.VMEM((3, BG), jnp.int32),
            pltpu.VMEM((3, BG, C), jnp.float32),
            pltpu.VMEM((NWORK, 128), jnp.int32),
            pltpu.SMEM((NWORK,), jnp.int32),
            pltpu.SMEM((_MAXB + 8,), jnp.int32),
            pltpu.SMEM((_MAXB + 8,), jnp.int32),
            pltpu.SemaphoreType.DMA,
            pltpu.SemaphoreType.DMA,
            pltpu.SemaphoreType.DMA,
            pltpu.SemaphoreType.DMA,
            pltpu.SemaphoreType.DMA,
            pltpu.SemaphoreType.DMA,
        ],
        compiler_params=pltpu.CompilerParams(needs_layout_passes=False, use_tc_tiling_on_sc=False),
    )
    def segmax(h_hbm, binned_hbm, counts_hbm, m_hbm,
               acc_v, pk_v, idx_v, rows_v, cnt_v, cnt_s, desc_s, lim_s,
               sa0, sa1, sa2, sg0, sg1, sg2):
        w = lax.axis_index("s") * 2 + lax.axis_index("c")
        pltpu.sync_copy(counts_hbm, cnt_v)
        ninf = jnp.full((16,), -jnp.inf, dtype=jnp.float32)
        iota = lax.iota(jnp.int32, 16)
        sas = [sa0, sa1, sa2]
        sgs = [sg0, sg1, sg2]

        def issue_pk(i, slot):
            off = pl.multiple_of(desc_s[i], 8)
            pltpu.async_copy(binned_hbm.at[pl.ds(off, BG)],
                             pk_v.at[slot], sas[slot])

        def wait_pk(slot):
            pltpu.make_async_copy(binned_hbm.at[pl.ds(0, BG)],
                                  pk_v.at[slot], sas[slot]).wait()

        def unpack_and_gather(slot):
            def up_body(i, c3):
                v = pk_v[slot, pl.ds(i * 16, 16)]
                idx_v[slot, pl.ds(i * 16, 16)] = jnp.minimum(
                    lax.shift_right_logical(v, 10), N_NODES - 1)
                return c3

            lax.fori_loop(0, BG // 16, up_body, 0)
            pltpu.async_copy(h_hbm.at[idx_v.at[slot]],
                             rows_v.at[slot], sgs[slot])

        def wait_gather(slot):
            pltpu.make_async_copy(h_hbm.at[idx_v.at[slot]],
                                  rows_v.at[slot], sgs[slot]).wait()

        def rmw(i, slot):
            lim = lim_s[i]
            lim_vec = jnp.full((16,), lim, dtype=jnp.int32)

            def g_body(g, c4):
                pk = pk_v[slot, pl.ds(g * 16, 16)]
                valid = (g * 16 + iota) < lim_vec
                base16 = jnp.where(
                    valid, jnp.bitwise_and(pk, 1023) * C, RPB * C)
                for l in range(16):
                    base = base16[l]
                    for cc in range(C // 16):
                        o = base + cc * 16
                        acc_v[pl.ds(o, 16)] = jnp.maximum(
                            acc_v[pl.ds(o, 16)],
                            rows_v[slot, g * 16 + l, pl.ds(cc * 16, 16)])
                return c4

            lax.fori_loop(0, BG // 16, g_body, 0)

        for r in range(4):
            b = r * NWORK + w

            @pl.when(b < NB)
            def _():
                bvec = jnp.full((16,), b, dtype=jnp.int32)
                ca = plsc.load_gather(cnt_v, [iota, bvec])
                cb = plsc.load_gather(cnt_v, [iota + 16, bvec])
                for l in range(16):
                    cnt_s[l] = ca[l]
                    cnt_s[16 + l] = cb[l]

                def init_body(i, carry):
                    acc_v[pl.ds(i * 16, 16)] = ninf
                    return carry

                lax.fori_loop(0, RPB * C // 16, init_body, 0)

                def sched_w2(w2, j):
                    nw = cnt_s[w2]
                    nbatch = lax.shift_right_logical(nw + BG - 1, 7)

                    def sched_s(s, j2):
                        desc_s[j2] = w2 * SEG + b * CAP + s * BG
                        lim_s[j2] = jnp.minimum(BG, nw - s * BG)
                        return j2 + 1

                    return lax.fori_loop(0, nbatch, sched_s, j)

                j = lax.fori_loop(0, NWORK, sched_w2, 0)

                def pad_body(p, carry):
                    desc_s[j + p] = b * CAP
                    lim_s[j + p] = 0
                    return carry

                lax.fori_loop(0, 6, pad_body, 0)
                nbt = lax.div(j + 2, 3) * 3

                issue_pk(0, 0)
                issue_pk(1, 1)
                wait_pk(0)
                unpack_and_gather(0)

                def pipe_t(t, carry):
                    for k in range(3):
                        i = t * 3 + k
                        issue_pk(i + 2, (k + 2) % 3)
                        wait_pk((k + 1) % 3)
                        unpack_and_gather((k + 1) % 3)
                        wait_gather(k)
                        rmw(i, k)
                    return carry

                lax.fori_loop(0, lax.div(nbt, 3), pipe_t, 0)
                wait_gather(0)
                wait_pk(1)

                pltpu.sync_copy(acc_v.at[pl.ds(0, RPB * C)],
                                m_hbm.at[pl.ds(b * RPB * C, RPB * C)])

    return segmax


_segmax16 = _make_segmax(16)
_segmax64 = _make_segmax(64)

_BLK = 1000


def _tc_layer_body(m_ref, h_ref, w_ref, b_ref, o_ref):
    m = m_ref[...]
    agg = jnp.where(m > -jnp.inf, m - h_ref[...], 0.0)
    y = jnp.dot(agg, w_ref[...], preferred_element_type=jnp.float32)
    o_ref[...] = jnp.maximum(y + b_ref[...], 0.0)


def _tc_layer(m, h, W, b):
    cin, cout = W.shape
    return pl.pallas_call(
        _tc_layer_body,
        grid=(N_NODES // _BLK,),
        in_specs=[
            pl.BlockSpec((_BLK, cin), lambda i: (i, 0)),
            pl.BlockSpec((_BLK, cin), lambda i: (i, 0)),
            pl.BlockSpec((cin, cout), lambda i: (0, 0)),
            pl.BlockSpec((1, cout), lambda i: (0, 0)),
        ],
        out_specs=pl.BlockSpec((_BLK, cout), lambda i: (i, 0)),
        out_shape=jax.ShapeDtypeStruct((N_NODES, cout), jnp.float32),
    )(m, h, W, b.reshape(1, cout))


def _tc_final_body(m_ref, h_ref, w_ref, b_ref, wo_ref, bo_ref, o_ref):
    m = m_ref[...]
    agg = jnp.where(m > -jnp.inf, m - h_ref[...], 0.0)
    y = jnp.dot(agg, w_ref[...], preferred_element_type=jnp.float32)
    h3 = jnp.maximum(y + b_ref[...], 0.0)
    s = jnp.dot(h3, wo_ref[...], preferred_element_type=jnp.float32)
    o_ref[...] = jax.nn.sigmoid(s + bo_ref[...])


def _tc_final(m, h, W, b, Wout, bout):
    return pl.pallas_call(
        _tc_final_body,
        grid=(N_NODES // _BLK,),
        in_specs=[
            pl.BlockSpec((_BLK, 64), lambda i: (i, 0)),
            pl.BlockSpec((_BLK, 64), lambda i: (i, 0)),
            pl.BlockSpec((64, 64), lambda i: (0, 0)),
            pl.BlockSpec((1, 64), lambda i: (0, 0)),
            pl.BlockSpec((64, 1), lambda i: (0, 0)),
            pl.BlockSpec((1, 1), lambda i: (0, 0)),
        ],
        out_specs=pl.BlockSpec((_BLK, 1), lambda i: (i, 0)),
        out_shape=jax.ShapeDtypeStruct((N_NODES, 1), jnp.float32),
    )(m, h, W, b.reshape(1, 64), Wout, bout.reshape(1, 1))


def kernel(x, edge_index, W1, b1, W2, b2, W3, b3, Wout, bout):
    ei = edge_index.reshape(-1)
    x16 = jnp.pad(x, ((0, 0), (0, 13)))
    W1p = jnp.pad(W1, ((0, 13), (0, 0)))

    binned, counts = _bin_edges(ei)
    binned = binned.reshape(-1)

    m1 = _segmax16(x16, binned, counts).reshape(NPAD, 16)[:N_NODES]
    h1 = _tc_layer(m1, x16, W1p, b1)

    m2 = _segmax64(h1, binned, counts).reshape(NPAD, 64)[:N_NODES]
    h2 = _tc_layer(m2, h1, W2, b2)

    m3 = _segmax64(h2, binned, counts).reshape(NPAD, 64)[:N_NODES]
    out = _tc_final(m3, h2, W3, b3, Wout, bout)
    return out[:, 0]
